# Initial kernel scaffold; baseline (speedup 1.0000x reference)
import jax
import jax.numpy as jnp
from jax.experimental import pallas as pl


def kernel(x, edge_index, W1, b1, W2, b2, W3, b3):
    raise NotImplementedError("write your pallas kernel here")


# speedup vs baseline: 5.8747x; 5.8747x over previous
import functools

import jax
import jax.numpy as jnp
from jax import lax
from jax.experimental import pallas as pl
from jax.experimental.pallas import tpu as pltpu
from jax.experimental.pallas import tpu_sc as plsc

N = 10000
E = 160000
D = 256
DC = 64
B = 128
NBLK = 1280
EPAD = NBLK * B - E
NACC = 10240
RPT = NACC // 16
EPT = NBLK * B // 32

_mesh = functools.partial(
    plsc.VectorSubcoreMesh, core_axis_name="c", subcore_axis_name="s"
)


@functools.partial(
    pl.kernel,
    out_type=jax.ShapeDtypeStruct((32 * NACC,), jnp.float32),
    mesh=_mesh(),
    scratch_types=[
        pltpu.VMEM((EPT,), jnp.int32),
        pltpu.VMEM((NACC,), jnp.float32),
    ],
    compiler_params=pltpu.CompilerParams(needs_layout_passes=False),
)
def _sc_degree(col1d, out, cbuf, cnt):
    c = lax.axis_index("c")
    s = lax.axis_index("s")
    w = c * 16 + s
    pltpu.sync_copy(col1d.at[pl.ds(w * EPT, EPT)], cbuf)

    def zbody(i, carry):
        cnt[pl.ds(i * 16, 16)] = jnp.zeros((16,), jnp.float32)
        return carry

    lax.fori_loop(0, NACC // 16, zbody, 0)
    ones = jnp.ones((16,), jnp.float32)

    def body(i, carry):
        idx = cbuf[pl.ds(i * 16, 16)]
        plsc.addupdate_scatter(cnt, [idx], ones)
        return carry

    lax.fori_loop(0, EPT // 16, body, 0)
    pltpu.sync_copy(cnt, out.at[pl.ds(w * NACC, NACC)])


@functools.partial(
    pl.kernel,
    out_type=jax.ShapeDtypeStruct((NACC, D), jnp.float32),
    mesh=_mesh(),
    scratch_types=[
        pltpu.VMEM((B,), jnp.int32),
        pltpu.VMEM((B,), jnp.int32),
        pltpu.VMEM((B, 128), jnp.float32),
        pltpu.VMEM_SHARED((NACC, 128), jnp.float32),
        pltpu.SemaphoreType.DMA,
    ],
)
def _sc_prop256(ua, ub, row1d, col1d, zeros_hbm, out, rbuf, cbuf, gbuf, acc, sem):
    c = lax.axis_index("c")
    s = lax.axis_index("s")
    pltpu.sync_copy(
        zeros_hbm.at[pl.ds(s * RPT, RPT), :], acc.at[pl.ds(s * RPT, RPT), :]
    )
    plsc.subcore_barrier()
    base = s * (NBLK // 16)

    def body(i, carry):
        pltpu.sync_copy(row1d.at[pl.ds((base + i) * B, B)], rbuf)
        pltpu.sync_copy(col1d.at[pl.ds((base + i) * B, B)], cbuf)

        @pl.when(c == 0)
        def _():
            pltpu.async_copy(ua.at[rbuf], gbuf, sem).wait()

        @pl.when(c == 1)
        def _():
            pltpu.async_copy(ub.at[rbuf], gbuf, sem).wait()

        pltpu.sync_copy(gbuf, acc.at[cbuf], add=True)
        return carry

    lax.fori_loop(0, NBLK // 16, body, 0)
    plsc.subcore_barrier()
    pltpu.sync_copy(
        acc.at[pl.ds(s * RPT, RPT), :],
        out.at[pl.ds(s * RPT, RPT), pl.ds(c * 128, 128)],
    )


@functools.partial(
    pl.kernel,
    out_type=jax.ShapeDtypeStruct((2, NACC, 128), jnp.float32),
    mesh=_mesh(),
    scratch_types=[
        pltpu.VMEM((B,), jnp.int32),
        pltpu.VMEM((B,), jnp.int32),
        pltpu.VMEM((B, 128), jnp.float32),
        pltpu.VMEM_SHARED((NACC, 128), jnp.float32),
        pltpu.SemaphoreType.DMA,
    ],
)
def _sc_prop128(u, row1d, col1d, zeros_hbm, out, rbuf, cbuf, gbuf, acc, sem):
    c = lax.axis_index("c")
    s = lax.axis_index("s")
    pltpu.sync_copy(
        zeros_hbm.at[pl.ds(s * RPT, RPT), :], acc.at[pl.ds(s * RPT, RPT), :]
    )
    plsc.subcore_barrier()
    base = c * (NBLK // 2) + s * (NBLK // 32)

    def body(i, carry):
        pltpu.sync_copy(row1d.at[pl.ds((base + i) * B, B)], rbuf)
        pltpu.sync_copy(col1d.at[pl.ds((base + i) * B, B)], cbuf)
        pltpu.async_copy(u.at[rbuf], gbuf, sem).wait()
        pltpu.sync_copy(gbuf, acc.at[cbuf], add=True)
        return carry

    lax.fori_loop(0, NBLK // 32, body, 0)
    plsc.subcore_barrier()
    pltpu.sync_copy(
        acc.at[pl.ds(s * RPT, RPT), :], out.at[c, pl.ds(s * RPT, RPT), :]
    )


def _tc_scale_in(degm, x):
    def body(degm_ref, x_ref, dinv_ref, ua_ref, ub_ref):
        deg = jnp.sum(degm_ref[0:N, :], axis=1, keepdims=True) + 1.0
        dinv = lax.rsqrt(deg)
        u1 = x_ref[...] * dinv
        dinv_ref[...] = dinv
        ua_ref[...] = u1[:, :128]
        ub_ref[...] = u1[:, 128:]

    return pl.pallas_call(
        body,
        out_shape=[
            jax.ShapeDtypeStruct((N, 1), jnp.float32),
            jax.ShapeDtypeStruct((N, 128), jnp.float32),
            jax.ShapeDtypeStruct((N, 128), jnp.float32),
        ],
    )(degm, x)


def _tc_rescale256(s1, ua, ub, dinv):
    def body(s_ref, ua_ref, ub_ref, dinv_ref, oa_ref, ob_ref):
        d2 = dinv_ref[...] * dinv_ref[...]
        oa_ref[...] = (s_ref[0:N, 0:128] + ua_ref[...]) * d2
        ob_ref[...] = (s_ref[0:N, 128:256] + ub_ref[...]) * d2

    return pl.pallas_call(
        body,
        out_shape=[
            jax.ShapeDtypeStruct((N, 128), jnp.float32),
            jax.ShapeDtypeStruct((N, 128), jnp.float32),
        ],
    )(s1, ua, ub, dinv)


def _tc_mlp(s2, ua, ub, dinv, W1, b1, W2, b2, W3, b3):
    def body(s_ref, ua_ref, ub_ref, dinv_ref, w1_ref, b1_ref, w2_ref,
             b2_ref, w3_ref, b3_ref, u3_ref, c3_ref):
        dinv = dinv_ref[...]
        u2 = jnp.concatenate([ua_ref[...], ub_ref[...]], axis=1)
        z = (s_ref[0:N, :] + u2) * dinv
        h1 = jnp.maximum(
            jnp.dot(z, w1_ref[...], preferred_element_type=jnp.float32)
            + b1_ref[...],
            0.0,
        )
        M = jnp.dot(w2_ref[...], w3_ref[...], preferred_element_type=jnp.float32)
        g = jnp.dot(h1, M, preferred_element_type=jnp.float32)
        u3_ref[...] = jnp.concatenate(
            [g * dinv, jnp.zeros((N, 128 - DC), jnp.float32)], axis=1
        )
        c3_ref[...] = (
            jnp.dot(b2_ref[...], w3_ref[...], preferred_element_type=jnp.float32)
            + b3_ref[...]
        )

    return pl.pallas_call(
        body,
        out_shape=[
            jax.ShapeDtypeStruct((N, 128), jnp.float32),
            jax.ShapeDtypeStruct((1, DC), jnp.float32),
        ],
    )(s2, ua, ub, dinv, W1, b1, W2, b2, W3, b3)


def _tc_rescale64(p3, u3, dinv):
    def body(p_ref, u_ref, dinv_ref, o_ref):
        d2 = dinv_ref[...] * dinv_ref[...]
        o_ref[...] = (p_ref[0, 0:N, :] + p_ref[1, 0:N, :] + u_ref[...]) * d2

    return pl.pallas_call(
        body, out_shape=jax.ShapeDtypeStruct((N, 128), jnp.float32)
    )(p3, u3, dinv)


def _tc_final(p4, u4, dinv, c3):
    def body(p_ref, u_ref, dinv_ref, c3_ref, o_ref):
        acc = p_ref[0, 0:N, :] + p_ref[1, 0:N, :] + u_ref[...]
        o_ref[...] = acc[:, 0:DC] * dinv_ref[...] + c3_ref[...]

    return pl.pallas_call(
        body, out_shape=jax.ShapeDtypeStruct((N, DC), jnp.float32)
    )(p4, u4, dinv, c3)


def kernel(x, edge_index, W1, b1, W2, b2, W3, b3):
    row = edge_index[0].astype(jnp.int32)
    col = edge_index[1].astype(jnp.int32)
    row1d = jnp.concatenate([row, jnp.zeros((EPAD,), jnp.int32)])
    col1d = jnp.concatenate([col, jnp.full((EPAD,), N, jnp.int32)])
    zeros128 = jnp.zeros((NACC, 128), jnp.float32)
    b1r = b1.reshape(1, D)
    b2r = b2.reshape(1, D)
    b3r = b3.reshape(1, DC)

    degf = _sc_degree(col1d)
    degm = degf.reshape(32, NACC).T
    dinv, u1a, u1b = _tc_scale_in(degm, x)
    s1 = _sc_prop256(u1a, u1b, row1d, col1d, zeros128)
    u2a, u2b = _tc_rescale256(s1, u1a, u1b, dinv)
    s2 = _sc_prop256(u2a, u2b, row1d, col1d, zeros128)
    u3, c3 = _tc_mlp(s2, u2a, u2b, dinv, W1, b1r, W2, b2r, W3, b3r)
    p3 = _sc_prop128(u3, row1d, col1d, zeros128)
    u4 = _tc_rescale64(p3, u3, dinv)
    p4 = _sc_prop128(u4, row1d, col1d, zeros128)
    out = _tc_final(p4, u4, dinv, c3)
    return out


# speedup vs baseline: 7.2238x; 1.2297x over previous
import functools

import jax
import jax.numpy as jnp
from jax import lax
from jax.experimental import pallas as pl
from jax.experimental.pallas import tpu as pltpu
from jax.experimental.pallas import tpu_sc as plsc

N = 10000
E = 160000
D = 256
DC = 64
B = 128
NBLK = 1280
EPAD = NBLK * B - E
NACC = 10240
RPT = NACC // 16
EPT = NBLK * B // 32
NBUF = 4

_mesh = functools.partial(
    plsc.VectorSubcoreMesh, core_axis_name="c", subcore_axis_name="s"
)


@functools.partial(
    pl.kernel,
    out_type=jax.ShapeDtypeStruct((32 * NACC,), jnp.float32),
    mesh=_mesh(),
    scratch_types=[
        pltpu.VMEM((EPT,), jnp.int32),
        pltpu.VMEM((NACC,), jnp.float32),
    ],
    compiler_params=pltpu.CompilerParams(needs_layout_passes=False),
)
def _sc_degree(col1d, out, cbuf, cnt):
    c = lax.axis_index("c")
    s = lax.axis_index("s")
    w = c * 16 + s
    pltpu.sync_copy(col1d.at[pl.ds(w * EPT, EPT)], cbuf)

    def zbody(i, carry):
        cnt[pl.ds(i * 16, 16)] = jnp.zeros((16,), jnp.float32)
        return carry

    lax.fori_loop(0, NACC // 16, zbody, 0)
    ones = jnp.ones((16,), jnp.float32)

    def body(i, carry):
        idx = cbuf[pl.ds(i * 16, 16)]
        plsc.addupdate_scatter(cnt, [idx], ones)
        return carry

    lax.fori_loop(0, EPT // 16, body, 0)
    pltpu.sync_copy(cnt, out.at[pl.ds(w * NACC, NACC)])


def _make_prop(nbt, feature_split):
    phases = nbt // 40
    if feature_split:
        out_type = jax.ShapeDtypeStruct((NACC, D), jnp.float32)
    else:
        out_type = jax.ShapeDtypeStruct((2, NACC, 128), jnp.float32)

    @functools.partial(
        pl.kernel,
        out_type=out_type,
        mesh=_mesh(),
        scratch_types=[
            pltpu.VMEM((48, B), jnp.int32),
            pltpu.VMEM((48, B), jnp.int32),
            pltpu.VMEM((2, B, 128), jnp.float32),
            pltpu.VMEM_SHARED((NACC, 128), jnp.float32),
            pltpu.SemaphoreType.DMA,
            pltpu.SemaphoreType.DMA,
            pltpu.SemaphoreType.DMA,
            pltpu.SemaphoreType.DMA,
        ],
    )
    def k(us, rowS, col2d, zeros_hbm, out, ridx, cidx, gbuf, acc,
          sg0, sg1, ss0, ss1):
        c = lax.axis_index("c")
        s = lax.axis_index("s")
        sg = (sg0, sg1)
        ss = (ss0, ss1)
        pltpu.sync_copy(
            zeros_hbm.at[pl.ds(s * RPT, RPT), :], acc.at[pl.ds(s * RPT, RPT), :]
        )
        plsc.subcore_barrier()

        for p in range(phases):
            if feature_split:
                blk0 = s * nbt + p * 40
                pltpu.sync_copy(rowS.at[c, pl.ds(blk0, 48), :], ridx)
            else:
                blk0 = c * (NBLK // 2) + s * nbt + p * 40
                pltpu.sync_copy(rowS.at[0, pl.ds(blk0, 48), :], ridx)
            pltpu.sync_copy(col2d.at[pl.ds(blk0, 48), :], cidx)
            for b in (0, 1):
                pltpu.async_copy(us.at[ridx.at[b]], gbuf.at[b], sg[b])

            def body(jo, carry):
                for b in (0, 1):
                    j = 2 * jo + b
                    pltpu.make_async_copy(
                        us.at[ridx.at[0]], gbuf.at[b], sg[b]
                    ).wait()
                    sd = pltpu.async_copy(
                        gbuf.at[b], acc.at[cidx.at[j]], ss[b], add=True
                    )
                    sd.wait()
                    pltpu.async_copy(us.at[ridx.at[j + 2]], gbuf.at[b], sg[b])
                return carry

            lax.fori_loop(0, 20, body, 0)
            for b in (0, 1):
                pltpu.make_async_copy(
                    us.at[ridx.at[0]], gbuf.at[b], sg[b]
                ).wait()
        plsc.subcore_barrier()
        if feature_split:
            pltpu.sync_copy(
                acc.at[pl.ds(s * RPT, RPT), :],
                out.at[pl.ds(s * RPT, RPT), pl.ds(c * 128, 128)],
            )
        else:
            pltpu.sync_copy(
                acc.at[pl.ds(s * RPT, RPT), :], out.at[c, pl.ds(s * RPT, RPT), :]
            )

    return k


_sc_prop256 = _make_prop(NBLK // 16, True)
_sc_prop128 = _make_prop(NBLK // 32, False)


def _tc_scale_in(degm, x):
    def body(degm_ref, x_ref, dinv_ref, us_ref):
        deg = jnp.sum(degm_ref[0:N, :], axis=1, keepdims=True) + 1.0
        dinv = lax.rsqrt(deg)
        u1 = x_ref[...] * dinv
        dinv_ref[...] = dinv
        us_ref[0:N, :] = u1[:, :128]
        us_ref[N : 2 * N, :] = u1[:, 128:]

    return pl.pallas_call(
        body,
        out_shape=[
            jax.ShapeDtypeStruct((N, 1), jnp.float32),
            jax.ShapeDtypeStruct((2 * N, 128), jnp.float32),
        ],
    )(degm, x)


def _tc_rescale256(s1, us, dinv):
    def body(s_ref, us_ref, dinv_ref, o_ref):
        d2 = dinv_ref[...] * dinv_ref[...]
        o_ref[0:N, :] = (s_ref[0:N, 0:128] + us_ref[0:N, :]) * d2
        o_ref[N : 2 * N, :] = (s_ref[0:N, 128:256] + us_ref[N : 2 * N, :]) * d2

    return pl.pallas_call(
        body, out_shape=jax.ShapeDtypeStruct((2 * N, 128), jnp.float32)
    )(s1, us, dinv)


def _tc_mlp(s2, us, dinv, W1, b1, W2, b2, W3, b3):
    def body(s_ref, us_ref, dinv_ref, w1_ref, b1_ref, w2_ref,
             b2_ref, w3_ref, b3_ref, u3_ref, c3_ref):
        dinv = dinv_ref[...]
        u2 = jnp.concatenate(
            [us_ref[0:N, :], us_ref[N : 2 * N, :]], axis=1
        )
        z = (s_ref[0:N, :] + u2) * dinv
        h1 = jnp.maximum(
            jnp.dot(z, w1_ref[...], preferred_element_type=jnp.float32)
            + b1_ref[...],
            0.0,
        )
        M = jnp.dot(w2_ref[...], w3_ref[...], preferred_element_type=jnp.float32)
        g = jnp.dot(h1, M, preferred_element_type=jnp.float32)
        u3_ref[...] = jnp.concatenate(
            [g * dinv, jnp.zeros((N, 128 - DC), jnp.float32)], axis=1
        )
        c3_ref[...] = (
            jnp.dot(b2_ref[...], w3_ref[...], preferred_element_type=jnp.float32)
            + b3_ref[...]
        )

    return pl.pallas_call(
        body,
        out_shape=[
            jax.ShapeDtypeStruct((N, 128), jnp.float32),
            jax.ShapeDtypeStruct((1, DC), jnp.float32),
        ],
    )(s2, us, dinv, W1, b1, W2, b2, W3, b3)


def _tc_rescale64(p3, u3, dinv):
    def body(p_ref, u_ref, dinv_ref, o_ref):
        d2 = dinv_ref[...] * dinv_ref[...]
        o_ref[...] = (p_ref[0, 0:N, :] + p_ref[1, 0:N, :] + u_ref[...]) * d2

    return pl.pallas_call(
        body, out_shape=jax.ShapeDtypeStruct((N, 128), jnp.float32)
    )(p3, u3, dinv)


def _tc_final(p4, u4, dinv, c3):
    def body(p_ref, u_ref, dinv_ref, c3_ref, o_ref):
        acc = p_ref[0, 0:N, :] + p_ref[1, 0:N, :] + u_ref[...]
        o_ref[...] = acc[:, 0:DC] * dinv_ref[...] + c3_ref[...]

    return pl.pallas_call(
        body, out_shape=jax.ShapeDtypeStruct((N, DC), jnp.float32)
    )(p4, u4, dinv, c3)


def kernel(x, edge_index, W1, b1, W2, b2, W3, b3):
    row = edge_index[0].astype(jnp.int32)
    col = edge_index[1].astype(jnp.int32)
    row2d = jnp.concatenate(
        [row, jnp.zeros((EPAD + 8 * B,), jnp.int32)]).reshape(NBLK + 8, B)
    col1d = jnp.concatenate([col, jnp.full((EPAD,), N, jnp.int32)])
    col2d = jnp.concatenate(
        [col1d, jnp.zeros((8 * B,), jnp.int32)]).reshape(NBLK + 8, B)
    rowS = jnp.stack([row2d, row2d + N])
    zeros128 = jnp.zeros((NACC, 128), jnp.float32)
    b1r = b1.reshape(1, D)
    b2r = b2.reshape(1, D)
    b3r = b3.reshape(1, DC)

    degf = _sc_degree(col1d)
    degm = degf.reshape(32, NACC).T
    dinv, u1s = _tc_scale_in(degm, x)
    s1 = _sc_prop256(u1s, rowS, col2d, zeros128)
    u2s = _tc_rescale256(s1, u1s, dinv)
    s2 = _sc_prop256(u2s, rowS, col2d, zeros128)
    u3, c3 = _tc_mlp(s2, u2s, dinv, W1, b1r, W2, b2r, W3, b3r)
    p3 = _sc_prop128(u3, rowS, col2d, zeros128)
    u4 = _tc_rescale64(p3, u3, dinv)
    p4 = _sc_prop128(u4, rowS, col2d, zeros128)
    out = _tc_final(p4, u4, dinv, c3)
    return out


# speedup vs baseline: 7.2310x; 1.0010x over previous
import functools

import jax
import jax.numpy as jnp
from jax import lax
from jax.experimental import pallas as pl
from jax.experimental.pallas import tpu as pltpu
from jax.experimental.pallas import tpu_sc as plsc

N = 10000
E = 160000
D = 256
DC = 64
B = 128
NBLK = 1280
EPAD = NBLK * B - E
NACC = 10240
RPT = NACC // 16
EPT = NBLK * B // 32
NBUF = 4

_mesh = functools.partial(
    plsc.VectorSubcoreMesh, core_axis_name="c", subcore_axis_name="s"
)


@functools.partial(
    pl.kernel,
    out_type=jax.ShapeDtypeStruct((32 * NACC,), jnp.float32),
    mesh=_mesh(),
    scratch_types=[
        pltpu.VMEM((EPT,), jnp.int32),
        pltpu.VMEM((NACC,), jnp.float32),
    ],
    compiler_params=pltpu.CompilerParams(needs_layout_passes=False),
)
def _sc_degree(col1d, out, cbuf, cnt):
    c = lax.axis_index("c")
    s = lax.axis_index("s")
    w = c * 16 + s
    pltpu.sync_copy(col1d.at[pl.ds(w * EPT, EPT)], cbuf)

    def zbody(i, carry):
        cnt[pl.ds(i * 16, 16)] = jnp.zeros((16,), jnp.float32)
        return carry

    lax.fori_loop(0, NACC // 16, zbody, 0)
    ones = jnp.ones((16,), jnp.float32)

    def body(i, carry):
        idx = cbuf[pl.ds(i * 16, 16)]
        plsc.addupdate_scatter(cnt, [idx], ones)
        return carry

    lax.fori_loop(0, EPT // 16, body, 0)
    pltpu.sync_copy(cnt, out.at[pl.ds(w * NACC, NACC)])


def _make_prop(nbt, feature_split):
    phases = nbt // 40
    if feature_split:
        out_type = jax.ShapeDtypeStruct((NACC, D), jnp.float32)
    else:
        out_type = jax.ShapeDtypeStruct((2, NACC, 128), jnp.float32)

    @functools.partial(
        pl.kernel,
        out_type=out_type,
        mesh=_mesh(),
        scratch_types=[
            pltpu.VMEM((48, B), jnp.int32),
            pltpu.VMEM((48, B), jnp.int32),
            pltpu.VMEM((2, B, 128), jnp.float32),
            pltpu.VMEM_SHARED((NACC, 128), jnp.float32),
            pltpu.SemaphoreType.DMA,
            pltpu.SemaphoreType.DMA,
            pltpu.SemaphoreType.DMA,
            pltpu.SemaphoreType.DMA,
        ],
    )
    def k(us, rowS, col2d, zeros_hbm, out, ridx, cidx, gbuf, acc,
          sg0, sg1, ss0, ss1):
        c = lax.axis_index("c")
        s = lax.axis_index("s")
        sg = (sg0, sg1)
        ss = (ss0, ss1)
        pltpu.sync_copy(
            zeros_hbm.at[pl.ds(s * RPT, RPT), :], acc.at[pl.ds(s * RPT, RPT), :]
        )
        plsc.subcore_barrier()

        for p in range(phases):
            if feature_split:
                blk0 = s * nbt + p * 40
                pltpu.sync_copy(rowS.at[c, pl.ds(blk0, 48), :], ridx)
            else:
                blk0 = c * (NBLK // 2) + s * nbt + p * 40
                pltpu.sync_copy(rowS.at[0, pl.ds(blk0, 48), :], ridx)
            pltpu.sync_copy(col2d.at[pl.ds(blk0, 48), :], cidx)
            for b in (0, 1):
                pltpu.async_copy(us.at[ridx.at[b]], gbuf.at[b], sg[b])

            def body(jo, carry):
                for b in (0, 1):
                    j = 2 * jo + b
                    pltpu.make_async_copy(
                        us.at[ridx.at[0]], gbuf.at[b], sg[b]
                    ).wait()
                    sd = pltpu.async_copy(
                        gbuf.at[b], acc.at[cidx.at[j]], ss[b], add=True
                    )
                    sd.wait()
                    pltpu.async_copy(us.at[ridx.at[j + 2]], gbuf.at[b], sg[b])
                return carry

            lax.fori_loop(0, 20, body, 0)
            for b in (0, 1):
                pltpu.make_async_copy(
                    us.at[ridx.at[0]], gbuf.at[b], sg[b]
                ).wait()
        plsc.subcore_barrier()
        if feature_split:
            pltpu.sync_copy(
                acc.at[pl.ds(s * RPT, RPT), :],
                out.at[pl.ds(s * RPT, RPT), pl.ds(c * 128, 128)],
            )
        else:
            pltpu.sync_copy(
                acc.at[pl.ds(s * RPT, RPT), :], out.at[c, pl.ds(s * RPT, RPT), :]
            )

    return k


_sc_prop256 = _make_prop(NBLK // 16, True)
_sc_prop128 = _make_prop(NBLK // 32, False)


def _tc_scale_in(degm, x):
    def body(degm_ref, x_ref, dinv_ref, us_ref):
        deg = jnp.sum(degm_ref[0:N, :], axis=1, keepdims=True) + 1.0
        dinv = lax.rsqrt(deg)
        u1 = x_ref[...] * dinv
        dinv_ref[...] = dinv
        us_ref[0:N, :] = u1[:, :128]
        us_ref[N : 2 * N, :] = u1[:, 128:]

    return pl.pallas_call(
        body,
        out_shape=[
            jax.ShapeDtypeStruct((N, 1), jnp.float32),
            jax.ShapeDtypeStruct((2 * N, 128), jnp.float32),
        ],
    )(degm, x)


def _tc_rescale256(s1, us, dinv):
    def body(s_ref, us_ref, dinv_ref, o_ref):
        d2 = dinv_ref[...] * dinv_ref[...]
        o_ref[0:N, :] = (s_ref[0:N, 0:128] + us_ref[0:N, :]) * d2
        o_ref[N : 2 * N, :] = (s_ref[0:N, 128:256] + us_ref[N : 2 * N, :]) * d2

    return pl.pallas_call(
        body, out_shape=jax.ShapeDtypeStruct((2 * N, 128), jnp.float32)
    )(s1, us, dinv)


def _tc_mlp(s2, us, dinv, W1, b1, W2, b2, W3, b3):
    def body(s_ref, us_ref, dinv_ref, w1_ref, b1_ref, w2_ref,
             b2_ref, w3_ref, b3_ref, u3_ref, c3_ref):
        dinv = dinv_ref[...]
        u2 = jnp.concatenate(
            [us_ref[0:N, :], us_ref[N : 2 * N, :]], axis=1
        )
        z = (s_ref[0:N, :] + u2) * dinv
        h1 = jnp.maximum(
            jnp.dot(z, w1_ref[...], preferred_element_type=jnp.float32)
            + b1_ref[...],
            0.0,
        )
        M = jnp.dot(w2_ref[...], w3_ref[...], preferred_element_type=jnp.float32)
        g = jnp.dot(h1, M, preferred_element_type=jnp.float32)
        u3_ref[...] = jnp.concatenate(
            [g * dinv, jnp.zeros((N, 128 - DC), jnp.float32)], axis=1
        )
        c3_ref[...] = (
            jnp.dot(b2_ref[...], w3_ref[...], preferred_element_type=jnp.float32)
            + b3_ref[...]
        )

    return pl.pallas_call(
        body,
        out_shape=[
            jax.ShapeDtypeStruct((N, 128), jnp.float32),
            jax.ShapeDtypeStruct((1, DC), jnp.float32),
        ],
    )(s2, us, dinv, W1, b1, W2, b2, W3, b3)


def _tc_rescale64(p3, u3, dinv):
    def body(p_ref, u_ref, dinv_ref, o_ref):
        d2 = dinv_ref[...] * dinv_ref[...]
        o_ref[...] = (p_ref[0, 0:N, :] + p_ref[1, 0:N, :] + u_ref[...]) * d2

    return pl.pallas_call(
        body, out_shape=jax.ShapeDtypeStruct((N, 128), jnp.float32)
    )(p3, u3, dinv)


def _tc_final(p4, u4, dinv, c3):
    def body(p_ref, u_ref, dinv_ref, c3_ref, o_ref):
        acc = p_ref[0, 0:N, :] + p_ref[1, 0:N, :] + u_ref[...]
        o_ref[...] = acc[:, 0:DC] * dinv_ref[...] + c3_ref[...]

    return pl.pallas_call(
        body, out_shape=jax.ShapeDtypeStruct((N, DC), jnp.float32)
    )(p4, u4, dinv, c3)


def kernel(x, edge_index, W1, b1, W2, b2, W3, b3):
    row = edge_index[0].astype(jnp.int32)
    col = edge_index[1].astype(jnp.int32)
    row2d = jnp.concatenate(
        [row, jnp.zeros((EPAD + 8 * B,), jnp.int32)]).reshape(NBLK + 8, B)
    padcol = N + (jnp.arange(EPAD, dtype=jnp.int32) % (NACC - N))
    col1d = jnp.concatenate([col, padcol])
    col2d = jnp.concatenate(
        [col1d, jnp.zeros((8 * B,), jnp.int32)]).reshape(NBLK + 8, B)
    rowS = jnp.stack([row2d, row2d + N])
    zeros128 = jnp.zeros((NACC, 128), jnp.float32)
    b1r = b1.reshape(1, D)
    b2r = b2.reshape(1, D)
    b3r = b3.reshape(1, DC)

    degf = _sc_degree(col1d)
    degm = degf.reshape(32, NACC).T
    dinv, u1s = _tc_scale_in(degm, x)
    s1 = _sc_prop256(u1s, rowS, col2d, zeros128)
    u2s = _tc_rescale256(s1, u1s, dinv)
    s2 = _sc_prop256(u2s, rowS, col2d, zeros128)
    u3, c3 = _tc_mlp(s2, u2s, dinv, W1, b1r, W2, b2r, W3, b3r)
    p3 = _sc_prop128(u3, rowS, col2d, zeros128)
    u4 = _tc_rescale64(p3, u3, dinv)
    p4 = _sc_prop128(u4, rowS, col2d, zeros128)
    out = _tc_final(p4, u4, dinv, c3)
    return out


# speedup vs baseline: 7.5880x; 1.0494x over previous
import functools

import jax
import jax.numpy as jnp
from jax import lax
from jax.experimental import pallas as pl
from jax.experimental.pallas import tpu as pltpu
from jax.experimental.pallas import tpu_sc as plsc

N = 10000
E = 160000
D = 256
DC = 64
B = 128
NBLK = 1280
EPAD = NBLK * B - E
NACC = 10240
RPT = NACC // 16
EPT = NBLK * B // 32
NBUF = 4

_mesh = functools.partial(
    plsc.VectorSubcoreMesh, core_axis_name="c", subcore_axis_name="s"
)


@functools.partial(
    pl.kernel,
    out_type=jax.ShapeDtypeStruct((32 * NACC,), jnp.float32),
    mesh=_mesh(),
    scratch_types=[
        pltpu.VMEM((EPT,), jnp.int32),
        pltpu.VMEM((NACC,), jnp.float32),
    ],
    compiler_params=pltpu.CompilerParams(needs_layout_passes=False),
)
def _sc_degree(col1d, out, cbuf, cnt):
    c = lax.axis_index("c")
    s = lax.axis_index("s")
    w = c * 16 + s
    pltpu.sync_copy(col1d.at[pl.ds(w * EPT, EPT)], cbuf)

    def zbody(i, carry):
        cnt[pl.ds(i * 16, 16)] = jnp.zeros((16,), jnp.float32)
        return carry

    lax.fori_loop(0, NACC // 16, zbody, 0)
    ones = jnp.ones((16,), jnp.float32)

    def body(i, carry):
        idx = cbuf[pl.ds(i * 16, 16)]
        plsc.addupdate_scatter(cnt, [idx], ones)
        return carry

    lax.fori_loop(0, EPT // 16, body, 0)
    pltpu.sync_copy(cnt, out.at[pl.ds(w * NACC, NACC)])


NB = 32
CHB = NBLK * B // 32


@functools.partial(
    pl.kernel,
    out_type=jax.ShapeDtypeStruct((NBLK * B,), jnp.int32),
    mesh=_mesh(),
    scratch_types=[
        pltpu.VMEM((CHB,), jnp.int32),
        pltpu.VMEM((CHB,), jnp.int32),
        pltpu.VMEM((CHB,), jnp.int32),
        pltpu.VMEM((NB,), jnp.int32),
    ],
    compiler_params=pltpu.CompilerParams(needs_layout_passes=False),
)
def _sc_binplace(row1d, col1d, out, rbuf, cbuf, packb, cur):
    c = lax.axis_index("c")
    s = lax.axis_index("s")
    w = c * 16 + s
    lane = lax.broadcasted_iota(jnp.int32, (16,), 0)
    zero16 = jnp.zeros((16,), jnp.int32)
    one16 = jnp.ones((16,), jnp.int32)

    pltpu.sync_copy(row1d.at[pl.ds(w * CHB, CHB)], rbuf)
    pltpu.sync_copy(col1d.at[pl.ds(w * CHB, CHB)], cbuf)
    cur[pl.ds(0, 16)] = zero16
    cur[pl.ds(16, 16)] = zero16

    def count(i, carry):
        b16 = rbuf[pl.ds(i * 16, 16)] >> 9
        fullcnt = zero16
        for m in range(16):
            fullcnt = fullcnt + jnp.where(b16 == b16[m], one16, zero16)
        g = plsc.load_gather(cur, [b16])
        plsc.store_scatter(cur, [b16], g + fullcnt)
        return carry

    lax.fori_loop(0, CHB // 16, count, 0)

    carry0 = jnp.int32(0)
    for h in (0, 1):
        tot = cur[pl.ds(h * 16, 16)]
        cs = plsc.cumsum(tot)
        cur[pl.ds(h * 16, 16)] = cs - tot + carry0
        carry0 = carry0 + cs[15]

    def place(i, carry):
        r16 = rbuf[pl.ds(i * 16, 16)]
        c16 = cbuf[pl.ds(i * 16, 16)]
        b16 = r16 >> 9
        fullcnt = zero16
        rank = zero16
        for m in range(16):
            eq = b16 == b16[m]
            fullcnt = fullcnt + jnp.where(eq, one16, zero16)
            rank = rank + jnp.where(eq & (lane > m), one16, zero16)
        g = plsc.load_gather(cur, [b16])
        plsc.store_scatter(cur, [b16], g + fullcnt)
        plsc.store_scatter(packb, [g + rank], r16 | (c16 << 14))
        return carry

    lax.fori_loop(0, CHB // 16, place, 0)
    pltpu.sync_copy(packb, out.at[pl.ds(w * CHB, CHB)])


def _make_prop(nbt, feature_split):
    phases = nbt // 40
    if feature_split:
        out_type = jax.ShapeDtypeStruct((NACC, D), jnp.float32)
    else:
        out_type = jax.ShapeDtypeStruct((2, NACC, 128), jnp.float32)

    @functools.partial(
        pl.kernel,
        out_type=out_type,
        mesh=_mesh(),
        scratch_types=[
            pltpu.VMEM((48, B), jnp.int32),
            pltpu.VMEM((48, B), jnp.int32),
            pltpu.VMEM((2, B, 128), jnp.float32),
            pltpu.VMEM_SHARED((NACC, 128), jnp.float32),
            pltpu.SemaphoreType.DMA,
            pltpu.SemaphoreType.DMA,
            pltpu.SemaphoreType.DMA,
            pltpu.SemaphoreType.DMA,
        ],
    )
    def k(us, rowS, col2d, zeros_hbm, out, ridx, cidx, gbuf, acc,
          sg0, sg1, ss0, ss1):
        c = lax.axis_index("c")
        s = lax.axis_index("s")
        sg = (sg0, sg1)
        ss = (ss0, ss1)
        pltpu.sync_copy(
            zeros_hbm.at[pl.ds(s * RPT, RPT), :], acc.at[pl.ds(s * RPT, RPT), :]
        )
        plsc.subcore_barrier()

        for p in range(phases):
            if feature_split:
                blk0 = s * nbt + p * 40
                pltpu.sync_copy(rowS.at[c, pl.ds(blk0, 48), :], ridx)
            else:
                blk0 = c * (NBLK // 2) + s * nbt + p * 40
                pltpu.sync_copy(rowS.at[0, pl.ds(blk0, 48), :], ridx)
            pltpu.sync_copy(col2d.at[pl.ds(blk0, 48), :], cidx)
            for b in (0, 1):
                pltpu.async_copy(us.at[ridx.at[b]], gbuf.at[b], sg[b])

            def body(jo, carry):
                for b in (0, 1):
                    j = 2 * jo + b
                    pltpu.make_async_copy(
                        us.at[ridx.at[0]], gbuf.at[b], sg[b]
                    ).wait()
                    sd = pltpu.async_copy(
                        gbuf.at[b], acc.at[cidx.at[j]], ss[b], add=True
                    )
                    sd.wait()
                    pltpu.async_copy(us.at[ridx.at[j + 2]], gbuf.at[b], sg[b])
                return carry

            lax.fori_loop(0, 20, body, 0)
            for b in (0, 1):
                pltpu.make_async_copy(
                    us.at[ridx.at[0]], gbuf.at[b], sg[b]
                ).wait()
        plsc.subcore_barrier()
        if feature_split:
            pltpu.sync_copy(
                acc.at[pl.ds(s * RPT, RPT), :],
                out.at[pl.ds(s * RPT, RPT), pl.ds(c * 128, 128)],
            )
        else:
            pltpu.sync_copy(
                acc.at[pl.ds(s * RPT, RPT), :], out.at[c, pl.ds(s * RPT, RPT), :]
            )

    return k


_sc_prop256 = _make_prop(NBLK // 16, True)
_sc_prop128 = _make_prop(NBLK // 32, False)


def _tc_scale_in(degm, x):
    def body(degm_ref, x_ref, dinv_ref, us_ref):
        deg = jnp.sum(degm_ref[0:N, :], axis=1, keepdims=True) + 1.0
        dinv = lax.rsqrt(deg)
        u1 = x_ref[...] * dinv
        dinv_ref[...] = dinv
        us_ref[0:N, :] = u1[:, :128]
        us_ref[N : 2 * N, :] = u1[:, 128:]

    return pl.pallas_call(
        body,
        out_shape=[
            jax.ShapeDtypeStruct((N, 1), jnp.float32),
            jax.ShapeDtypeStruct((2 * N, 128), jnp.float32),
        ],
    )(degm, x)


def _tc_rescale256(s1, us, dinv):
    def body(s_ref, us_ref, dinv_ref, o_ref):
        d2 = dinv_ref[...] * dinv_ref[...]
        o_ref[0:N, :] = (s_ref[0:N, 0:128] + us_ref[0:N, :]) * d2
        o_ref[N : 2 * N, :] = (s_ref[0:N, 128:256] + us_ref[N : 2 * N, :]) * d2

    return pl.pallas_call(
        body, out_shape=jax.ShapeDtypeStruct((2 * N, 128), jnp.float32)
    )(s1, us, dinv)


def _tc_mlp(s2, us, dinv, W1, b1, W2, b2, W3, b3):
    def body(s_ref, us_ref, dinv_ref, w1_ref, b1_ref, w2_ref,
             b2_ref, w3_ref, b3_ref, u3_ref, c3_ref):
        dinv = dinv_ref[...]
        u2 = jnp.concatenate(
            [us_ref[0:N, :], us_ref[N : 2 * N, :]], axis=1
        )
        z = (s_ref[0:N, :] + u2) * dinv
        h1 = jnp.maximum(
            jnp.dot(z, w1_ref[...], preferred_element_type=jnp.float32)
            + b1_ref[...],
            0.0,
        )
        M = jnp.dot(w2_ref[...], w3_ref[...], preferred_element_type=jnp.float32)
        g = jnp.dot(h1, M, preferred_element_type=jnp.float32)
        u3_ref[...] = jnp.concatenate(
            [g * dinv, jnp.zeros((N, 128 - DC), jnp.float32)], axis=1
        )
        c3_ref[...] = (
            jnp.dot(b2_ref[...], w3_ref[...], preferred_element_type=jnp.float32)
            + b3_ref[...]
        )

    return pl.pallas_call(
        body,
        out_shape=[
            jax.ShapeDtypeStruct((N, 128), jnp.float32),
            jax.ShapeDtypeStruct((1, DC), jnp.float32),
        ],
    )(s2, us, dinv, W1, b1, W2, b2, W3, b3)


def _tc_rescale64(p3, u3, dinv):
    def body(p_ref, u_ref, dinv_ref, o_ref):
        d2 = dinv_ref[...] * dinv_ref[...]
        o_ref[...] = (p_ref[0, 0:N, :] + p_ref[1, 0:N, :] + u_ref[...]) * d2

    return pl.pallas_call(
        body, out_shape=jax.ShapeDtypeStruct((N, 128), jnp.float32)
    )(p3, u3, dinv)


def _tc_final(p4, u4, dinv, c3):
    def body(p_ref, u_ref, dinv_ref, c3_ref, o_ref):
        acc = p_ref[0, 0:N, :] + p_ref[1, 0:N, :] + u_ref[...]
        o_ref[...] = acc[:, 0:DC] * dinv_ref[...] + c3_ref[...]

    return pl.pallas_call(
        body, out_shape=jax.ShapeDtypeStruct((N, DC), jnp.float32)
    )(p4, u4, dinv, c3)


def kernel(x, edge_index, W1, b1, W2, b2, W3, b3):
    row = edge_index[0].astype(jnp.int32)
    col = edge_index[1].astype(jnp.int32)
    row1d = jnp.concatenate([row, jnp.zeros((EPAD,), jnp.int32)])
    padcol = N + (jnp.arange(EPAD, dtype=jnp.int32) % (NACC - N))
    col1d = jnp.concatenate([col, padcol])
    packP = _sc_binplace(row1d, col1d)
    rowP = packP & jnp.int32(16383)
    colP = packP >> 14
    row2d = jnp.concatenate(
        [rowP, jnp.zeros((8 * B,), jnp.int32)]).reshape(NBLK + 8, B)
    col2d = jnp.concatenate(
        [colP, jnp.zeros((8 * B,), jnp.int32)]).reshape(NBLK + 8, B)
    rowS = jnp.stack([row2d, row2d + N])
    zeros128 = jnp.zeros((NACC, 128), jnp.float32)
    b1r = b1.reshape(1, D)
    b2r = b2.reshape(1, D)
    b3r = b3.reshape(1, DC)

    degf = _sc_degree(col1d)
    degm = degf.reshape(32, NACC).T
    dinv, u1s = _tc_scale_in(degm, x)
    s1 = _sc_prop256(u1s, rowS, col2d, zeros128)
    u2s = _tc_rescale256(s1, u1s, dinv)
    s2 = _sc_prop256(u2s, rowS, col2d, zeros128)
    u3, c3 = _tc_mlp(s2, u2s, dinv, W1, b1r, W2, b2r, W3, b3r)
    p3 = _sc_prop128(u3, rowS, col2d, zeros128)
    u4 = _tc_rescale64(p3, u3, dinv)
    p4 = _sc_prop128(u4, rowS, col2d, zeros128)
    out = _tc_final(p4, u4, dinv, c3)
    return out
